# baseline (speedup 1.0000x reference)
import functools

import jax
import jax.numpy as jnp
from jax import lax
from jax.experimental import pallas as pl
from jax.experimental.pallas import tpu as pltpu
from jax.experimental.pallas import tpu_sc as plsc

_LANES = 16
_NTILES = 32
_BLK_E = 4000
_VPB = _BLK_E // _LANES
_UNROLL = 5


def _make_sc_partials(n_elems: int):
    assert n_elems % _BLK_E == 0
    n_blocks = n_elems // _BLK_E
    blocks_per_tile = -(-n_blocks // _NTILES)

    mesh = plsc.VectorSubcoreMesh(core_axis_name="c", subcore_axis_name="s")

    @functools.partial(
        pl.kernel,
        mesh=mesh,
        out_type=jax.ShapeDtypeStruct((_NTILES, 4 * _LANES), jnp.float32),
        compiler_params=pltpu.CompilerParams(
            needs_layout_passes=False, use_tc_tiling_on_sc=False,
            disable_bounds_checks=True, disable_semaphore_checks=True),
        scratch_types=(
            [pltpu.VMEM((_BLK_E,), jnp.float32)] * 6
            + [pltpu.VMEM((_BLK_E,), jnp.int32)] * 3
            + [pltpu.VMEM((4 * _LANES,), jnp.float32)]
            + [pltpu.SemaphoreType.DMA] * 9
        ),
    )
    def sc_partials(p01_hbm, lab_hbm, out_hbm,
                    a0, a1, a2, b0, b1, b2, l0, l1, l2, stag,
                    sa0, sa1, sa2, sb0, sb1, sb2, sl0, sl1, sl2):
        wid = lax.axis_index("c") * 16 + lax.axis_index("s")
        abufs, bbufs, lbufs = (a0, a1, a2), (b0, b1, b2), (l0, l1, l2)
        asems, bsems, lsems = (sa0, sa1, sa2), (sb0, sb1, sb2), (sl0, sl1, sl2)

        zero = jnp.zeros((_LANES,), jnp.float32)

        _NBUF = 3

        def start_dma(i):
            g = jnp.minimum(wid + _NTILES * i, n_blocks - 1)
            ha = pltpu.async_copy(
                p01_hbm.at[pl.ds(g * _BLK_E, _BLK_E)],
                abufs[i % _NBUF], asems[i % _NBUF])
            hb = pltpu.async_copy(
                p01_hbm.at[pl.ds(n_elems + g * _BLK_E, _BLK_E)],
                bbufs[i % _NBUF], bsems[i % _NBUF])
            hl = pltpu.async_copy(
                lab_hbm.at[pl.ds(g * _BLK_E, _BLK_E)],
                lbufs[i % _NBUF], lsems[i % _NBUF])
            return ha, hb, hl

        def block_partials(abuf, bbuf, lbuf):
            zero4 = (zero, zero, zero, zero)

            def update(accs, k):
                s_p1l, s_p0, s_p0l, s_cnt = accs
                off = k * _LANES
                lv = lbuf[pl.ds(off, _LANES)]
                v0 = abuf[pl.ds(off, _LANES)]
                v1 = bbuf[pl.ds(off, _LANES)]
                lvf = lv.astype(jnp.float32)
                return (s_p1l + lvf * v1, s_p0 + v0,
                        s_p0l + lvf * v0, s_cnt + lvf)

            @plsc.parallel_loop(0, _VPB, step=2, unroll=_UNROLL,
                                carry=(zero4, zero4))
            def body(k, carry):
                ca, cb = carry
                return update(ca, k), update(cb, k + 1)

            ca, cb = body
            return tuple(a + b for a, b in zip(ca, cb))

        accs = [zero, zero, zero, zero]
        handles = [start_dma(j) for j in range(min(_NBUF, blocks_per_tile))]
        for i in range(blocks_per_tile):
            for h in handles[i % _NBUF]:
                h.wait()
            parts = block_partials(abufs[i % _NBUF], bbufs[i % _NBUF],
                                   lbufs[i % _NBUF])
            gate = jnp.where(wid + _NTILES * i < n_blocks,
                             jnp.float32(1.0), jnp.float32(0.0))
            accs = [a + gate * p for a, p in zip(accs, parts)]
            if i + _NBUF < blocks_per_tile:
                handles[i % _NBUF] = start_dma(i + _NBUF)

        for j, a in enumerate(accs):
            stag[pl.ds(j * _LANES, _LANES)] = a
        pltpu.sync_copy(stag, out_hbm.at[wid])

    return sc_partials


def kernel(pred, label):
    lab = label.reshape(-1).astype(jnp.int32)
    n = lab.shape[0]
    p01 = pred.reshape(n, 2).T.reshape(-1)

    parts = _make_sc_partials(n)(p01, lab)
    s = jnp.sum(parts.reshape(_NTILES, 4, _LANES), axis=(0, 2))
    sum_pos, sum_p0, sum_p0l, cnt_pos = s[0], s[1], s[2], s[3]
    sum_neg = sum_p0 - sum_p0l
    cnt_neg = jnp.float32(n) - cnt_pos
    loss_pos = jnp.where(cnt_pos > 0, -sum_pos / jnp.maximum(cnt_pos, 1.0), 0.0)
    loss_neg = jnp.where(cnt_neg > 0, -sum_neg / jnp.maximum(cnt_neg, 1.0), 0.0)
    return loss_pos * 0.5 + loss_neg * 0.5


# speedup vs baseline: 1.0162x; 1.0162x over previous
import functools

import jax
import jax.numpy as jnp
from jax import lax
from jax.experimental import pallas as pl
from jax.experimental.pallas import tpu as pltpu
from jax.experimental.pallas import tpu_sc as plsc

_LANES = 16
_NTILES = 32
_BLK_E = 8000
_VPB = _BLK_E // _LANES
_UNROLL = 5


def _make_sc_partials(n_elems: int):
    assert n_elems % _BLK_E == 0
    n_blocks = n_elems // _BLK_E
    blocks_per_tile = -(-n_blocks // _NTILES)

    mesh = plsc.VectorSubcoreMesh(core_axis_name="c", subcore_axis_name="s")

    @functools.partial(
        pl.kernel,
        mesh=mesh,
        out_type=jax.ShapeDtypeStruct((_NTILES, 4 * _LANES), jnp.float32),
        compiler_params=pltpu.CompilerParams(
            needs_layout_passes=False, use_tc_tiling_on_sc=False,
            disable_bounds_checks=True, disable_semaphore_checks=True),
        scratch_types=(
            [pltpu.VMEM((_BLK_E,), jnp.float32)] * 6
            + [pltpu.VMEM((_BLK_E,), jnp.int32)] * 3
            + [pltpu.VMEM((4 * _LANES,), jnp.float32)]
            + [pltpu.SemaphoreType.DMA] * 9
        ),
    )
    def sc_partials(p01_hbm, lab_hbm, out_hbm,
                    a0, a1, a2, b0, b1, b2, l0, l1, l2, stag,
                    sa0, sa1, sa2, sb0, sb1, sb2, sl0, sl1, sl2):
        wid = lax.axis_index("c") * 16 + lax.axis_index("s")
        abufs, bbufs, lbufs = (a0, a1, a2), (b0, b1, b2), (l0, l1, l2)
        asems, bsems, lsems = (sa0, sa1, sa2), (sb0, sb1, sb2), (sl0, sl1, sl2)

        zero = jnp.zeros((_LANES,), jnp.float32)

        _NBUF = 3

        def start_dma(i):
            g = jnp.minimum(wid + _NTILES * i, n_blocks - 1)
            ha = pltpu.async_copy(
                p01_hbm.at[pl.ds(g * _BLK_E, _BLK_E)],
                abufs[i % _NBUF], asems[i % _NBUF])
            hb = pltpu.async_copy(
                p01_hbm.at[pl.ds(n_elems + g * _BLK_E, _BLK_E)],
                bbufs[i % _NBUF], bsems[i % _NBUF])
            hl = pltpu.async_copy(
                lab_hbm.at[pl.ds(g * _BLK_E, _BLK_E)],
                lbufs[i % _NBUF], lsems[i % _NBUF])
            return ha, hb, hl

        def block_partials(abuf, bbuf, lbuf):
            zero4 = (zero, zero, zero, zero)

            def update(accs, k):
                s_p1l, s_p0, s_p0l, s_cnt = accs
                off = k * _LANES
                lv = lbuf[pl.ds(off, _LANES)]
                v0 = abuf[pl.ds(off, _LANES)]
                v1 = bbuf[pl.ds(off, _LANES)]
                lvf = lv.astype(jnp.float32)
                return (s_p1l + lvf * v1, s_p0 + v0,
                        s_p0l + lvf * v0, s_cnt + lvf)

            @plsc.parallel_loop(0, _VPB, step=2, unroll=_UNROLL,
                                carry=(zero4, zero4))
            def body(k, carry):
                ca, cb = carry
                return update(ca, k), update(cb, k + 1)

            ca, cb = body
            return tuple(a + b for a, b in zip(ca, cb))

        accs = [zero, zero, zero, zero]
        handles = [start_dma(j) for j in range(min(_NBUF, blocks_per_tile))]
        for i in range(blocks_per_tile):
            for h in handles[i % _NBUF]:
                h.wait()
            parts = block_partials(abufs[i % _NBUF], bbufs[i % _NBUF],
                                   lbufs[i % _NBUF])
            gate = jnp.where(wid + _NTILES * i < n_blocks,
                             jnp.float32(1.0), jnp.float32(0.0))
            accs = [a + gate * p for a, p in zip(accs, parts)]
            if i + _NBUF < blocks_per_tile:
                handles[i % _NBUF] = start_dma(i + _NBUF)

        for j, a in enumerate(accs):
            stag[pl.ds(j * _LANES, _LANES)] = a
        pltpu.sync_copy(stag, out_hbm.at[wid])

    return sc_partials


def kernel(pred, label):
    lab = label.reshape(-1).astype(jnp.int32)
    n = lab.shape[0]
    p01 = pred.reshape(n, 2).T.reshape(-1)

    parts = _make_sc_partials(n)(p01, lab)
    s = jnp.sum(parts.reshape(_NTILES, 4, _LANES), axis=(0, 2))
    sum_pos, sum_p0, sum_p0l, cnt_pos = s[0], s[1], s[2], s[3]
    sum_neg = sum_p0 - sum_p0l
    cnt_neg = jnp.float32(n) - cnt_pos
    loss_pos = jnp.where(cnt_pos > 0, -sum_pos / jnp.maximum(cnt_pos, 1.0), 0.0)
    loss_neg = jnp.where(cnt_neg > 0, -sum_neg / jnp.maximum(cnt_neg, 1.0), 0.0)
    return loss_pos * 0.5 + loss_neg * 0.5
